# Initial kernel scaffold; baseline (speedup 1.0000x reference)
import jax
import jax.numpy as jnp
from jax.experimental import pallas as pl


def kernel(x, edge_index, W1, b1, W2, b2):
    raise NotImplementedError("write your pallas kernel here")


# speedup vs baseline: 10.0558x; 10.0558x over previous
import dataclasses

import jax
import jax.numpy as jnp
from jax import lax
from jax.experimental import pallas as pl
from jax.experimental.pallas import tpu as pltpu
from jax.experimental.pallas import tpu_sc as plsc

N = 10000
E = 320000
D = 128
DH = D // 2
C = 128
NCH = 160
EPW = NCH * C
EPAD = 16 * EPW
NPAD = 10240
DUMMY = N
RB = 512
GRID = NPAD // RB
SPT = NPAD // 16
DCH = NCH // 2


def _deg_body(dst_hbm, degp_hbm, idx_v, degp_v):
    c = lax.axis_index("core")
    s = lax.axis_index("subcore")
    w = c * 16 + s
    z16 = jnp.zeros((16,), jnp.float32)

    @pl.loop(0, NPAD // 16)
    def _(i):
        degp_v[pl.ds(i * 16, 16)] = z16

    pltpu.sync_copy(dst_hbm.at[s, pl.ds(c * DCH, DCH)], idx_v)
    one16 = jnp.ones((16,), jnp.float32)

    @pl.loop(0, DCH)
    def _(r):
        @pl.loop(0, C // 16)
        def _(k):
            iv = idx_v[r, pl.ds(k * 16, 16)]
            plsc.addupdate_scatter(degp_v, [iv], one16)

    pltpu.sync_copy(degp_v, degp_hbm.at[w])


def _agg_body(glo_hbm, ghi_hbm, src_hbm, dst_hbm, p_hbm, idxs_v, idxd_v,
              rows_v, wb_v, acc_sh, sem):
    c = lax.axis_index("core")
    s = lax.axis_index("subcore")
    pltpu.sync_copy(src_hbm.at[s], idxs_v)
    pltpu.sync_copy(dst_hbm.at[s], idxd_v)

    z16 = jnp.zeros((16,), jnp.float32)

    @pl.loop(0, SPT)
    def _(r):
        @pl.loop(0, DH // 16)
        def _(k):
            wb_v[r, pl.ds(k * 16, 16)] = z16

    pltpu.sync_copy(wb_v, acc_sh.at[pl.ds(s * SPT, SPT)])
    plsc.subcore_barrier()

    def _loop(g_ref):
        @pl.loop(0, NCH)
        def _(j):
            pltpu.async_copy(g_ref.at[idxs_v.at[j]], rows_v, sem).wait()
            pltpu.sync_copy(rows_v, acc_sh.at[idxd_v.at[j]], add=True)

    @pl.when(c == 0)
    def _():
        _loop(glo_hbm)

    @pl.when(c == 1)
    def _():
        _loop(ghi_hbm)

    plsc.subcore_barrier()
    pltpu.sync_copy(acc_sh.at[pl.ds(s * SPT, SPT)], wb_v)
    pltpu.sync_copy(wb_v, p_hbm.at[c, pl.ds(s * SPT, SPT)])


def _sc_calls():
    mesh = plsc.VectorSubcoreMesh(core_axis_name="core",
                                  subcore_axis_name="subcore")
    cp = pltpu.CompilerParams(use_tc_tiling_on_sc=False)
    if "needs_layout_passes" in pltpu.CompilerParams.__dataclass_fields__:
        cp = dataclasses.replace(cp, needs_layout_passes=False)
    deg = pl.kernel(
        _deg_body,
        out_type=jax.ShapeDtypeStruct((32, NPAD), jnp.float32),
        mesh=mesh,
        compiler_params=cp,
        scratch_types=[pltpu.VMEM((DCH, C), jnp.int32),
                       pltpu.VMEM((NPAD,), jnp.float32)],
    )
    agg = pl.kernel(
        _agg_body,
        out_type=jax.ShapeDtypeStruct((2, NPAD, DH), jnp.float32),
        mesh=mesh,
        compiler_params=cp,
        scratch_types=[pltpu.VMEM((NCH, C), jnp.int32),
                       pltpu.VMEM((NCH, C), jnp.int32),
                       pltpu.VMEM((C, DH), jnp.float32),
                       pltpu.VMEM((SPT, DH), jnp.float32),
                       pltpu.VMEM_SHARED((NPAD, DH), jnp.float32),
                       pltpu.SemaphoreType.DMA],
    )
    return deg, agg


def _mm1_body(x_ref, w_ref, degp_ref, glo_ref, ghi_ref, dinv_ref):
    deg = jnp.sum(degp_ref[...], axis=0) + 1.0
    dinv = lax.rsqrt(deg)[:, None]
    h = jnp.dot(x_ref[...], w_ref[...], preferred_element_type=jnp.float32)
    g = h * dinv
    glo_ref[...] = g[:, :DH]
    ghi_ref[...] = g[:, DH:]
    dinv_ref[...] = dinv


def _mm2_body(glo_ref, ghi_ref, p0_ref, p1_ref, dinv_ref, b1_ref, w2_ref,
              xh_ref, g2lo_ref, g2hi_ref):
    dinv = dinv_ref[...]
    y = jnp.concatenate([glo_ref[...] + p0_ref[0], ghi_ref[...] + p1_ref[0]],
                        axis=1)
    hrow = y * dinv + b1_ref[...]
    xh_ref[...] = hrow
    r = jnp.maximum(hrow, 0.0)
    h2 = jnp.dot(r, w2_ref[...], preferred_element_type=jnp.float32)
    g2 = h2 * dinv
    g2lo_ref[...] = g2[:, :DH]
    g2hi_ref[...] = g2[:, DH:]


def _out_body(g2lo_ref, g2hi_ref, q0_ref, q1_ref, dinv_ref, b2_ref, o_ref):
    y = jnp.concatenate([g2lo_ref[...] + q0_ref[0],
                         g2hi_ref[...] + q1_ref[0]], axis=1)
    o_ref[...] = y * dinv_ref[...] + b2_ref[...]


def _row_spec(r_dim, c_dim):
    return pl.BlockSpec((r_dim, c_dim), lambda r: (r, 0))


def _tc_calls():
    full = pl.BlockSpec((D, D), lambda r: (0, 0))
    bias = pl.BlockSpec((1, D), lambda r: (0, 0))
    pslice = [pl.BlockSpec((1, RB, DH), lambda r: (0, r, 0)),
              pl.BlockSpec((1, RB, DH), lambda r: (1, r, 0))]
    mm1 = pl.pallas_call(
        _mm1_body,
        grid=(GRID,),
        in_specs=[_row_spec(RB, D), full,
                  pl.BlockSpec((32, RB), lambda r: (0, r))],
        out_specs=[_row_spec(RB, DH), _row_spec(RB, DH), _row_spec(RB, 1)],
        out_shape=[jax.ShapeDtypeStruct((NPAD, DH), jnp.float32),
                   jax.ShapeDtypeStruct((NPAD, DH), jnp.float32),
                   jax.ShapeDtypeStruct((NPAD, 1), jnp.float32)],
    )
    mm2 = pl.pallas_call(
        _mm2_body,
        grid=(GRID,),
        in_specs=[_row_spec(RB, DH), _row_spec(RB, DH)] + pslice
                 + [_row_spec(RB, 1), bias, full],
        out_specs=[_row_spec(RB, D), _row_spec(RB, DH), _row_spec(RB, DH)],
        out_shape=[jax.ShapeDtypeStruct((N, D), jnp.float32),
                   jax.ShapeDtypeStruct((NPAD, DH), jnp.float32),
                   jax.ShapeDtypeStruct((NPAD, DH), jnp.float32)],
    )
    out = pl.pallas_call(
        _out_body,
        grid=(GRID,),
        in_specs=[_row_spec(RB, DH), _row_spec(RB, DH)] + pslice
                 + [_row_spec(RB, 1), bias],
        out_specs=_row_spec(RB, D),
        out_shape=jax.ShapeDtypeStruct((N, D), jnp.float32),
    )
    return mm1, mm2, out


def kernel(x, edge_index, W1, b1, W2, b2):
    src = edge_index[0]
    dst = edge_index[1]
    pad = EPAD - E
    srcp = jnp.concatenate(
        [src, jnp.zeros((pad,), jnp.int32)]).reshape(16, NCH, C)
    dstp = jnp.concatenate(
        [dst, jnp.full((pad,), DUMMY, jnp.int32)]).reshape(16, NCH, C)
    x_pad = jnp.pad(x, ((0, NPAD - N), (0, 0)))
    b1r = b1.reshape(1, D)
    b2r = b2.reshape(1, D)

    deg_call, agg_call = _sc_calls()
    mm1, mm2, out_call = _tc_calls()

    degp = deg_call(dstp)
    g1lo, g1hi, dinv = mm1(x_pad, W1, degp)
    p = agg_call(g1lo, g1hi, srcp, dstp)
    xh, g2lo, g2hi = mm2(g1lo, g1hi, p, p, dinv, b1r, W2)
    q = agg_call(g2lo, g2hi, srcp, dstp)
    h2 = out_call(g2lo, g2hi, q, q, dinv, b2r)
    return (h2, xh)


# speedup vs baseline: 11.8383x; 1.1773x over previous
import dataclasses

import jax
import jax.numpy as jnp
from jax import lax
from jax.experimental import pallas as pl
from jax.experimental.pallas import tpu as pltpu
from jax.experimental.pallas import tpu_sc as plsc

N = 10000
E = 320000
D = 128
DH = D // 2
C = 64
NCH = 320
EPW = NCH * C
EPAD = 16 * EPW
NPAD = 10240
DUMMY = N
RB = 512
GRID = NPAD // RB
SPT = NPAD // 16
DCH = NCH // 2


def _deg_body(dst_hbm, degp_hbm, idx_v, degp_v):
    c = lax.axis_index("core")
    s = lax.axis_index("subcore")
    w = c * 16 + s
    z16 = jnp.zeros((16,), jnp.float32)

    @pl.loop(0, NPAD // 16)
    def _(i):
        degp_v[pl.ds(i * 16, 16)] = z16

    pltpu.sync_copy(dst_hbm.at[s, pl.ds(c * DCH, DCH)], idx_v)
    one16 = jnp.ones((16,), jnp.float32)

    @pl.loop(0, DCH)
    def _(r):
        @pl.loop(0, C // 16)
        def _(k):
            iv = idx_v[r, pl.ds(k * 16, 16)]
            plsc.addupdate_scatter(degp_v, [iv], one16)

    pltpu.sync_copy(degp_v, degp_hbm.at[w])


NBUF = 2


def _agg_body(glo_hbm, ghi_hbm, src_hbm, dst_hbm, p_hbm, idxs_v, idxd_v,
              rows_v, wb_v, acc_sh, gs0, gs1):
    c = lax.axis_index("core")
    s = lax.axis_index("subcore")
    bufs = [rows_v.at[b] for b in range(NBUF)]
    gsems = [gs0, gs1]
    pltpu.sync_copy(src_hbm.at[s], idxs_v)
    pltpu.sync_copy(dst_hbm.at[s], idxd_v)

    z16 = jnp.zeros((16,), jnp.float32)

    @pl.loop(0, SPT)
    def _(r):
        @pl.loop(0, DH // 16)
        def _(k):
            wb_v[r, pl.ds(k * 16, 16)] = z16

    pltpu.sync_copy(wb_v, acc_sh.at[pl.ds(s * SPT, SPT)])
    plsc.subcore_barrier()

    def _loop(g_ref):
        def start_g(j, b):
            pltpu.async_copy(g_ref.at[idxs_v.at[j]], bufs[b], gsems[b])

        def wait_g(j, b):
            pltpu.make_async_copy(g_ref.at[idxs_v.at[j]], bufs[b],
                                  gsems[b]).wait()

        def scat(j, b):
            pltpu.sync_copy(bufs[b], acc_sh.at[idxd_v.at[j]], add=True)

        for b in range(NBUF):
            start_g(b, b)

        @pl.loop(0, NCH - NBUF, step=NBUF)
        def _(jo):
            for b in range(NBUF):
                j = jo + b
                wait_g(j, b)
                scat(j, b)
                start_g(j + NBUF, b)

        for b in range(NBUF):
            j = NCH - NBUF + b
            wait_g(j, b)
            scat(j, b)

    @pl.when(c == 0)
    def _():
        _loop(glo_hbm)

    @pl.when(c == 1)
    def _():
        _loop(ghi_hbm)

    plsc.subcore_barrier()
    pltpu.sync_copy(acc_sh.at[pl.ds(s * SPT, SPT)], wb_v)
    pltpu.sync_copy(wb_v, p_hbm.at[c, pl.ds(s * SPT, SPT)])


def _sc_calls():
    mesh = plsc.VectorSubcoreMesh(core_axis_name="core",
                                  subcore_axis_name="subcore")
    cp = pltpu.CompilerParams(use_tc_tiling_on_sc=False, internal_scratch_in_bytes=65536)
    if "needs_layout_passes" in pltpu.CompilerParams.__dataclass_fields__:
        cp = dataclasses.replace(cp, needs_layout_passes=False)
    deg = pl.kernel(
        _deg_body,
        out_type=jax.ShapeDtypeStruct((32, NPAD), jnp.float32),
        mesh=mesh,
        compiler_params=cp,
        scratch_types=[pltpu.VMEM((DCH, C), jnp.int32),
                       pltpu.VMEM((NPAD,), jnp.float32)],
    )
    agg = pl.kernel(
        _agg_body,
        out_type=jax.ShapeDtypeStruct((2, NPAD, DH), jnp.float32),
        mesh=mesh,
        compiler_params=cp,
        scratch_types=[pltpu.VMEM((NCH, C), jnp.int32),
                       pltpu.VMEM((NCH, C), jnp.int32),
                       pltpu.VMEM((NBUF, C, DH), jnp.float32),
                       pltpu.VMEM((SPT, DH), jnp.float32),
                       pltpu.VMEM_SHARED((NPAD, DH), jnp.float32)]
                      + [pltpu.SemaphoreType.DMA] * 2,
    )
    return deg, agg


def _mm1_body(x_ref, w_ref, degp_ref, glo_ref, ghi_ref, dinv_ref):
    deg = jnp.sum(degp_ref[...], axis=0) + 1.0
    dinv = lax.rsqrt(deg)[:, None]
    h = jnp.dot(x_ref[...], w_ref[...], preferred_element_type=jnp.float32)
    g = h * dinv
    glo_ref[...] = g[:, :DH]
    ghi_ref[...] = g[:, DH:]
    dinv_ref[...] = dinv


def _mm2_body(glo_ref, ghi_ref, p0_ref, p1_ref, dinv_ref, b1_ref, w2_ref,
              xh_ref, g2lo_ref, g2hi_ref):
    dinv = dinv_ref[...]
    y = jnp.concatenate([glo_ref[...] + p0_ref[0], ghi_ref[...] + p1_ref[0]],
                        axis=1)
    hrow = y * dinv + b1_ref[...]
    xh_ref[...] = hrow
    r = jnp.maximum(hrow, 0.0)
    h2 = jnp.dot(r, w2_ref[...], preferred_element_type=jnp.float32)
    g2 = h2 * dinv
    g2lo_ref[...] = g2[:, :DH]
    g2hi_ref[...] = g2[:, DH:]


def _out_body(g2lo_ref, g2hi_ref, q0_ref, q1_ref, dinv_ref, b2_ref, o_ref):
    y = jnp.concatenate([g2lo_ref[...] + q0_ref[0],
                         g2hi_ref[...] + q1_ref[0]], axis=1)
    o_ref[...] = y * dinv_ref[...] + b2_ref[...]


def _row_spec(r_dim, c_dim):
    return pl.BlockSpec((r_dim, c_dim), lambda r: (r, 0))


def _tc_calls():
    full = pl.BlockSpec((D, D), lambda r: (0, 0))
    bias = pl.BlockSpec((1, D), lambda r: (0, 0))
    pslice = [pl.BlockSpec((1, RB, DH), lambda r: (0, r, 0)),
              pl.BlockSpec((1, RB, DH), lambda r: (1, r, 0))]
    mm1 = pl.pallas_call(
        _mm1_body,
        grid=(GRID,),
        in_specs=[_row_spec(RB, D), full,
                  pl.BlockSpec((32, RB), lambda r: (0, r))],
        out_specs=[_row_spec(RB, DH), _row_spec(RB, DH), _row_spec(RB, 1)],
        out_shape=[jax.ShapeDtypeStruct((NPAD, DH), jnp.float32),
                   jax.ShapeDtypeStruct((NPAD, DH), jnp.float32),
                   jax.ShapeDtypeStruct((NPAD, 1), jnp.float32)],
    )
    mm2 = pl.pallas_call(
        _mm2_body,
        grid=(GRID,),
        in_specs=[_row_spec(RB, DH), _row_spec(RB, DH)] + pslice
                 + [_row_spec(RB, 1), bias, full],
        out_specs=[_row_spec(RB, D), _row_spec(RB, DH), _row_spec(RB, DH)],
        out_shape=[jax.ShapeDtypeStruct((N, D), jnp.float32),
                   jax.ShapeDtypeStruct((NPAD, DH), jnp.float32),
                   jax.ShapeDtypeStruct((NPAD, DH), jnp.float32)],
    )
    out = pl.pallas_call(
        _out_body,
        grid=(GRID,),
        in_specs=[_row_spec(RB, DH), _row_spec(RB, DH)] + pslice
                 + [_row_spec(RB, 1), bias],
        out_specs=_row_spec(RB, D),
        out_shape=jax.ShapeDtypeStruct((N, D), jnp.float32),
    )
    return mm1, mm2, out


def kernel(x, edge_index, W1, b1, W2, b2):
    src = edge_index[0]
    dst = edge_index[1]
    pad = EPAD - E
    srcp = jnp.concatenate(
        [src, jnp.zeros((pad,), jnp.int32)]).reshape(16, NCH, C)
    dstp = jnp.concatenate(
        [dst, jnp.full((pad,), DUMMY, jnp.int32)]).reshape(16, NCH, C)
    x_pad = jnp.pad(x, ((0, NPAD - N), (0, 0)))
    b1r = b1.reshape(1, D)
    b2r = b2.reshape(1, D)

    deg_call, agg_call = _sc_calls()
    mm1, mm2, out_call = _tc_calls()

    degp = deg_call(dstp)
    g1lo, g1hi, dinv = mm1(x_pad, W1, degp)
    p = agg_call(g1lo, g1hi, srcp, dstp)
    xh, g2lo, g2hi = mm2(g1lo, g1hi, p, p, dinv, b1r, W2)
    q = agg_call(g2lo, g2hi, srcp, dstp)
    h2 = out_call(g2lo, g2hi, q, q, dinv, b2r)
    return (h2, xh)


# speedup vs baseline: 12.0927x; 1.0215x over previous
import dataclasses

import jax
import jax.numpy as jnp
from jax import lax
from jax.experimental import pallas as pl
from jax.experimental.pallas import tpu as pltpu
from jax.experimental.pallas import tpu_sc as plsc

N = 10000
E = 320000
D = 128
DH = D // 2
C = 64
NCH = 320
EPW = NCH * C
EPAD = 16 * EPW
NPAD = 10240
DUMMY = N
RB = 512
GRID = NPAD // RB
SPT = NPAD // 16
DCH = NCH // 2
NBUF = 2
LAG = 1


def _deg_body(dst_hbm, degp_hbm, idx_v, degp_v):
    c = lax.axis_index("core")
    s = lax.axis_index("subcore")
    w = c * 16 + s
    z16 = jnp.zeros((16,), jnp.float32)

    @pl.loop(0, NPAD // 16)
    def _(i):
        degp_v[pl.ds(i * 16, 16)] = z16

    pltpu.sync_copy(dst_hbm.at[s, pl.ds(c * DCH, DCH)], idx_v)
    one16 = jnp.ones((16,), jnp.float32)

    @pl.loop(0, DCH)
    def _(r):
        @pl.loop(0, C // 16)
        def _(k):
            iv = idx_v[r, pl.ds(k * 16, 16)]
            plsc.addupdate_scatter(degp_v, [iv], one16)

    pltpu.sync_copy(degp_v, degp_hbm.at[w])


def _agg_body(glo_hbm, ghi_hbm, src_hbm, dst_hbm, p_hbm, idxs_v, idxd_v,
              rows_v, wb_v, acc_sh, gs0, gs1, gs2, gs3, ss0, ss1, ss2, ss3):
    c = lax.axis_index("core")
    s = lax.axis_index("subcore")
    bufs = [rows_v.at[b] for b in range(NBUF)]
    gsems = [gs0, gs1, gs2, gs3][:NBUF]
    ssems = [ss0, ss1, ss2, ss3][:NBUF]
    pltpu.sync_copy(src_hbm.at[s], idxs_v)
    pltpu.sync_copy(dst_hbm.at[s], idxd_v)

    z16 = jnp.zeros((16,), jnp.float32)

    @pl.loop(0, SPT)
    def _(r):
        @pl.loop(0, DH // 16)
        def _(k):
            wb_v[r, pl.ds(k * 16, 16)] = z16

    pltpu.sync_copy(wb_v, acc_sh.at[pl.ds(s * SPT, SPT)])
    plsc.subcore_barrier()

    def _loop(g_ref):
        def start_g(j, b):
            pltpu.async_copy(g_ref.at[idxs_v.at[j]], bufs[b], gsems[b])

        def wait_g(j, b):
            pltpu.make_async_copy(g_ref.at[idxs_v.at[j]], bufs[b],
                                  gsems[b]).wait()

        def start_s(j, b):
            pltpu.async_copy(bufs[b], acc_sh.at[idxd_v.at[j]], ssems[b],
                             add=True)

        def wait_s(j, b):
            pltpu.make_async_copy(bufs[b], acc_sh.at[idxd_v.at[j]],
                                  ssems[b]).wait()

        @pl.loop(0, NCH + NBUF, step=NBUF)
        def _(jo):
            for k in range(NBUF):
                j = jo + k
                b2 = (k - LAG) % NBUF

                @pl.when(j >= NBUF)
                def _():
                    wait_s(j - NBUF, k)

                @pl.when(j < NCH)
                def _():
                    start_g(j, k)

                @pl.when(jnp.logical_and(j >= LAG, j < NCH + LAG))
                def _():
                    wait_g(j - LAG, b2)
                    start_s(j - LAG, b2)

    @pl.when(c == 0)
    def _():
        _loop(glo_hbm)

    @pl.when(c == 1)
    def _():
        _loop(ghi_hbm)

    plsc.subcore_barrier()
    pltpu.sync_copy(acc_sh.at[pl.ds(s * SPT, SPT)], wb_v)
    pltpu.sync_copy(wb_v, p_hbm.at[c, pl.ds(s * SPT, SPT)])


def _sc_calls():
    mesh = plsc.VectorSubcoreMesh(core_axis_name="core",
                                  subcore_axis_name="subcore")
    cp = pltpu.CompilerParams(use_tc_tiling_on_sc=False)
    if "needs_layout_passes" in pltpu.CompilerParams.__dataclass_fields__:
        cp = dataclasses.replace(cp, needs_layout_passes=False)
    deg = pl.kernel(
        _deg_body,
        out_type=jax.ShapeDtypeStruct((32, NPAD), jnp.float32),
        mesh=mesh,
        compiler_params=cp,
        scratch_types=[pltpu.VMEM((DCH, C), jnp.int32),
                       pltpu.VMEM((NPAD,), jnp.float32)],
    )
    agg = pl.kernel(
        _agg_body,
        out_type=jax.ShapeDtypeStruct((2, NPAD, DH), jnp.float32),
        mesh=mesh,
        compiler_params=cp,
        scratch_types=[pltpu.VMEM((NCH, C), jnp.int32),
                       pltpu.VMEM((NCH, C), jnp.int32),
                       pltpu.VMEM((NBUF, C, DH), jnp.float32),
                       pltpu.VMEM((SPT, DH), jnp.float32),
                       pltpu.VMEM_SHARED((NPAD, DH), jnp.float32)]
                      + [pltpu.SemaphoreType.DMA] * 8,
    )
    return deg, agg


def _dinv_body(degp_ref, dinv_ref):
    deg = jnp.sum(degp_ref[...], axis=0) + 1.0
    dinv_ref[...] = lax.rsqrt(deg)[:, None]


def _mm_body(h_ref, w_ref, dinv_ref, flag_ref, glo_ref, ghi_ref):
    x = h_ref[...]
    x = jnp.where(flag_ref[0, 0] > 0.0, x, jnp.maximum(x, 0.0))
    g = jnp.dot(x, w_ref[...], preferred_element_type=jnp.float32)
    g = g * dinv_ref[...]
    glo_ref[...] = g[:, :DH]
    ghi_ref[...] = g[:, DH:]


def _merge_body(glo_ref, ghi_ref, p0_ref, p1_ref, dinv_ref, b_ref, h_ref):
    y = jnp.concatenate([glo_ref[...] + p0_ref[0], ghi_ref[...] + p1_ref[0]],
                        axis=1)
    h_ref[...] = y * dinv_ref[...] + b_ref[...]


def _row_spec(r_dim, c_dim):
    return pl.BlockSpec((r_dim, c_dim), lambda r: (r, 0))


def _tc_calls():
    full = pl.BlockSpec((D, D), lambda r: (0, 0))
    bias = pl.BlockSpec((1, D), lambda r: (0, 0))
    flag = pl.BlockSpec((1, 1), lambda r: (0, 0))
    pslice = [pl.BlockSpec((1, RB, DH), lambda r: (0, r, 0)),
              pl.BlockSpec((1, RB, DH), lambda r: (1, r, 0))]
    dinv = pl.pallas_call(
        _dinv_body,
        grid=(GRID,),
        in_specs=[pl.BlockSpec((32, RB), lambda r: (0, r))],
        out_specs=_row_spec(RB, 1),
        out_shape=jax.ShapeDtypeStruct((NPAD, 1), jnp.float32),
    )
    mm = pl.pallas_call(
        _mm_body,
        grid=(GRID,),
        in_specs=[_row_spec(RB, D), full, _row_spec(RB, 1), flag],
        out_specs=[_row_spec(RB, DH), _row_spec(RB, DH)],
        out_shape=[jax.ShapeDtypeStruct((NPAD, DH), jnp.float32),
                   jax.ShapeDtypeStruct((NPAD, DH), jnp.float32)],
    )
    merge = pl.pallas_call(
        _merge_body,
        grid=(GRID,),
        in_specs=[_row_spec(RB, DH), _row_spec(RB, DH)] + pslice
                 + [_row_spec(RB, 1), bias],
        out_specs=_row_spec(RB, D),
        out_shape=jax.ShapeDtypeStruct((NPAD, D), jnp.float32),
    )
    return dinv, mm, merge


def kernel(x, edge_index, W1, b1, W2, b2):
    src = edge_index[0]
    dst = edge_index[1]
    pad = EPAD - E
    srcp = jnp.concatenate(
        [src, jnp.zeros((pad,), jnp.int32)]).reshape(16, NCH, C)
    dstp = jnp.concatenate(
        [dst, jnp.full((pad,), DUMMY, jnp.int32)]).reshape(16, NCH, C)
    x_pad = jnp.pad(x, ((0, NPAD - N), (0, 0)))

    deg_call, agg_call = _sc_calls()
    dinv_call, mm_call, merge_call = _tc_calls()

    degp = deg_call(dstp)
    dinv = dinv_call(degp)

    Ws = jnp.stack([W1, W2])
    bs = jnp.stack([b1.reshape(1, D), b2.reshape(1, D)])
    flags = jnp.array([[[1.0]], [[0.0]]], jnp.float32)

    def body(h_prev, layer):
        W, b, flag = layer
        glo, ghi = mm_call(h_prev, W, dinv, flag)
        p = agg_call(glo, ghi, srcp, dstp)
        h = merge_call(glo, ghi, p, p, dinv, b)
        return h, h

    xh = None
    h = x_pad
    for i in range(2):
        h, _ = body(h, (Ws[i], bs[i], flags[i]))
        if i == 0:
            xh = h
    return (h[:N], xh[:N])
